# Initial kernel scaffold; baseline (speedup 1.0000x reference)
import jax
import jax.numpy as jnp
from jax.experimental import pallas as pl


def kernel(inp, objmask, AAidxs_tgts, W_obj, b_obj, W_act, b_act, W_ne, b_ne):
    raise NotImplementedError("write your pallas kernel here")


# speedup vs baseline: 1.9642x; 1.9642x over previous
import functools

import jax
import jax.numpy as jnp
from jax import lax
from jax.experimental import pallas as pl
from jax.experimental.pallas import tpu as pltpu
from jax.experimental.pallas import tpu_sc as plsc

B = 64
M = 32
D = 1024
NUM_ACT = 117
NUM_CLASSES = 600
N = M * NUM_ACT
ACT_PAD = 128
NE_PAD = 608
LANES = 16
CHUNKS = N // LANES
NW = 32
BPW = B // NW


def _tc_body(x_ref, mf_ref, mask_ref, wact_ref, bact_ref, wne_ref, bne_ref,
             act_ref, ne_ref):
    x = x_ref[...]
    act = jnp.dot(x, wact_ref[...], preferred_element_type=jnp.float32)
    act = act + bact_ref[...]
    mf = mf_ref[...]
    sel = (act > 0.0) & (mf.astype(jnp.int32) == 1)
    act_ref[...] = jnp.where(sel, act, 0.0)
    xm = x * mf
    pooled = jnp.sum(xm.reshape(B, M, D), axis=1)
    cnt = jnp.sum(mask_ref[...], axis=1, keepdims=True)
    pooled = pooled / cnt
    ne = jnp.dot(pooled, wne_ref[...], preferred_element_type=jnp.float32)
    ne_ref[...] = ne + bne_ref[...]


_tc_call = pl.pallas_call(
    _tc_body,
    out_shape=[
        jax.ShapeDtypeStruct((B * M, ACT_PAD), jnp.float32),
        jax.ShapeDtypeStruct((B, NE_PAD), jnp.float32),
    ],
)


@functools.partial(
    pl.kernel,
    mesh=plsc.VectorSubcoreMesh(core_axis_name="c", subcore_axis_name="s"),
    compiler_params=pltpu.CompilerParams(needs_layout_passes=False),
    out_type=jax.ShapeDtypeStruct((B, NE_PAD), jnp.float32),
    scratch_types=[
        pltpu.VMEM((N,), jnp.float32),
        pltpu.VMEM((N,), jnp.int32),
        pltpu.VMEM((NE_PAD,), jnp.float32),
        pltpu.VMEM((NE_PAD,), jnp.float32),
    ],
)
def _sc_scatter(vals_hbm, idx_hbm, ne_hbm, out_hbm, vals_v, idx_v, bins_v, ne_v):
    wid = lax.axis_index("s") * 2 + lax.axis_index("c")

    def per_batch(j, carry):
        b = wid * BPW + j
        pltpu.sync_copy(vals_hbm.at[b], vals_v)
        pltpu.sync_copy(idx_hbm.at[b], idx_v)
        pltpu.sync_copy(ne_hbm.at[b], ne_v)
        for i in range(NE_PAD // LANES):
            bins_v[pl.ds(i * LANES, LANES)] = jnp.zeros((LANES,), jnp.float32)

        def chunk(k, c):
            off = pl.multiple_of(k * LANES, LANES)
            v = vals_v[pl.ds(off, LANES)]
            ix = idx_v[pl.ds(off, LANES)]
            safe = jnp.where((ix >= 0) & (ix < NUM_CLASSES), ix, NE_PAD - 1)
            cur = plsc.load_gather(bins_v, [safe])
            new = jnp.maximum(cur, v)
            plsc.store_scatter(bins_v, [safe], new)
            chk = plsc.load_gather(bins_v, [safe])
            needi = jnp.where(chk < new, 1, 0)

            def w_cond(s):
                return jnp.any(s[0] != 0)

            def w_body(s):
                ndi, tgt = s
                nd = ndi != 0
                cur2 = plsc.load_gather(bins_v, [safe])
                upd = jnp.maximum(cur2, tgt)
                plsc.store_scatter(bins_v, [safe], upd, mask=nd)
                chk2 = plsc.load_gather(bins_v, [safe])
                ndi2 = jnp.where(nd & (chk2 < upd), 1, 0)
                return (ndi2, tgt)

            lax.while_loop(w_cond, w_body, (needi, new))
            return c

        lax.fori_loop(0, CHUNKS, chunk, 0)
        for i in range(NE_PAD // LANES):
            sl = pl.ds(i * LANES, LANES)
            seg = bins_v[sl]
            bins_v[sl] = jnp.where(seg != 0.0, seg, ne_v[sl])
        pltpu.sync_copy(bins_v, out_hbm.at[b])
        return carry

    lax.fori_loop(0, BPW, per_batch, 0)


def kernel(inp, objmask, AAidxs_tgts, W_obj, b_obj, W_act, b_act, W_ne, b_ne):
    del W_obj, b_obj
    x = inp.reshape(B * M, D)
    mf = objmask.reshape(B * M, 1)
    wact = jnp.pad(W_act, ((0, 0), (0, ACT_PAD - NUM_ACT)))
    bact = jnp.pad(b_act, (0, ACT_PAD - NUM_ACT)).reshape(1, ACT_PAD)
    wne = jnp.pad(W_ne, ((0, 0), (0, NE_PAD - NUM_CLASSES)))
    bne = jnp.pad(b_ne, (0, NE_PAD - NUM_CLASSES)).reshape(1, NE_PAD)
    act_pad, ne = _tc_call(x, mf, objmask, wact, bact, wne, bne)
    vals = act_pad.reshape(B, M, ACT_PAD)[:, :, :NUM_ACT].reshape(B, N)
    idx = AAidxs_tgts.astype(jnp.int32)
    out = _sc_scatter(vals, idx, ne)
    return out[:, :NUM_CLASSES]


# speedup vs baseline: 2.7217x; 1.3856x over previous
import functools

import jax
import jax.numpy as jnp
from jax import lax
from jax.experimental import pallas as pl
from jax.experimental.pallas import tpu as pltpu
from jax.experimental.pallas import tpu_sc as plsc

B = 64
M = 32
D = 1024
NUM_ACT = 117
NUM_CLASSES = 600
N = M * NUM_ACT
NE_PAD = 608
LANES = 16
CHUNKS = N // LANES
GROUPS = NE_PAD // LANES
WAYS = 16
NW = 32
BPW = B // NW


def _tc_body(x_ref, mf_ref, mask_ref, wact_ref, bact_ref, wne_ref, bne_ref,
             vals_ref, ne_ref):
    x = x_ref[...]
    act = jnp.dot(x, wact_ref[...], preferred_element_type=jnp.float32)
    act = act + bact_ref[...]
    mf = mf_ref[...]
    sel = (act > 0.0) & (mf.astype(jnp.int32) == 1)
    act = jnp.where(sel, act, 0.0)
    act3 = act.reshape(B, M, NUM_ACT)
    for m in range(M):
        vals_ref[:, pl.ds(m * NUM_ACT, NUM_ACT)] = act3[:, m, :]
    xm = x * mf
    pooled = jnp.sum(xm.reshape(B, M, D), axis=1)
    cnt = jnp.sum(mask_ref[...], axis=1, keepdims=True)
    pooled = pooled / cnt
    ne = jnp.dot(pooled, wne_ref[...], preferred_element_type=jnp.float32)
    ne_ref[:, pl.ds(0, NUM_CLASSES)] = ne + bne_ref[...]
    ne_ref[:, pl.ds(NUM_CLASSES, NE_PAD - NUM_CLASSES)] = jnp.zeros(
        (B, NE_PAD - NUM_CLASSES), jnp.float32)


_tc_call = pl.pallas_call(
    _tc_body,
    out_shape=[
        jax.ShapeDtypeStruct((B, N), jnp.float32),
        jax.ShapeDtypeStruct((B, NE_PAD), jnp.float32),
    ],
)


@functools.partial(
    pl.kernel,
    mesh=plsc.VectorSubcoreMesh(core_axis_name="c", subcore_axis_name="s"),
    compiler_params=pltpu.CompilerParams(needs_layout_passes=False),
    out_type=jax.ShapeDtypeStruct((B, NE_PAD), jnp.float32),
    scratch_types=[
        pltpu.VMEM((N,), jnp.float32),
        pltpu.VMEM((N,), jnp.float32),
        pltpu.VMEM((N,), jnp.int32),
        pltpu.VMEM((N,), jnp.int32),
        pltpu.VMEM((WAYS * NE_PAD,), jnp.float32),
        pltpu.VMEM((NE_PAD,), jnp.float32),
        pltpu.VMEM((NE_PAD,), jnp.float32),
        pltpu.VMEM((NE_PAD,), jnp.float32),
        pltpu.VMEM((NE_PAD,), jnp.float32),
        pltpu.SemaphoreType.DMA,
        pltpu.SemaphoreType.DMA,
        pltpu.SemaphoreType.DMA,
    ],
)
def _sc_scatter(vals_hbm, idx_hbm, ne_hbm, out_hbm,
                vals_v0, vals_v1, idx_v0, idx_v1, bins_v, ne_v0, ne_v1,
                out_v0, out_v1, in_sem0, in_sem1, out_sem):
    vals_vs = (vals_v0, vals_v1)
    idx_vs = (idx_v0, idx_v1)
    ne_vs = (ne_v0, ne_v1)
    out_vs = (out_v0, out_v1)
    wid = lax.axis_index("s") * 2 + lax.axis_index("c")
    b0 = wid * BPW
    way_off = lax.iota(jnp.int32, LANES) * NE_PAD
    zeros16 = jnp.zeros((LANES,), jnp.float32)

    in_sems = (in_sem0, in_sem1)
    copies = []
    for j in range(BPW):
        copies.append((
            pltpu.async_copy(vals_hbm.at[b0 + j], vals_vs[j], in_sems[j]),
            pltpu.async_copy(idx_hbm.at[b0 + j], idx_vs[j], in_sems[j]),
            pltpu.async_copy(ne_hbm.at[b0 + j], ne_vs[j], in_sems[j]),
        ))

    def zero_grp(i, c):
        off = pl.multiple_of(i * LANES, LANES)
        for w in range(WAYS):
            bins_v[pl.ds(w * NE_PAD + off, LANES)] = zeros16
        return c
    lax.fori_loop(0, GROUPS, zero_grp, 0)

    out_copies = []
    for j in range(BPW):
        for cp in copies[j]:
            cp.wait()

        vv, iv, nv, ov = vals_vs[j], idx_vs[j], ne_vs[j], out_vs[j]

        def chunk(k, c):
            off = pl.multiple_of(k * LANES, LANES)
            v = vv[pl.ds(off, LANES)]
            ix = iv[pl.ds(off, LANES)]
            safe = jnp.where((ix >= 0) & (ix < NUM_CLASSES), ix, NE_PAD - 1)
            addr = safe + way_off
            cur = plsc.load_gather(bins_v, [addr])
            plsc.store_scatter(bins_v, [addr], jnp.maximum(cur, v))
            return c
        lax.fori_loop(0, CHUNKS, chunk, 0)

        def combine(i, c):
            off = pl.multiple_of(i * LANES, LANES)
            seg = bins_v[pl.ds(off, LANES)]
            bins_v[pl.ds(off, LANES)] = zeros16
            for w in range(1, WAYS):
                seg = jnp.maximum(seg, bins_v[pl.ds(w * NE_PAD + off, LANES)])
                bins_v[pl.ds(w * NE_PAD + off, LANES)] = zeros16
            sl2 = pl.ds(off, LANES)
            ov[sl2] = jnp.where(seg != 0.0, seg, nv[sl2])
            return c
        lax.fori_loop(0, GROUPS, combine, 0)

        out_copies.append(pltpu.async_copy(ov, out_hbm.at[b0 + j], out_sem))
    for cp in out_copies:
        cp.wait()


def kernel(inp, objmask, AAidxs_tgts, W_obj, b_obj, W_act, b_act, W_ne, b_ne):
    del W_obj, b_obj
    x = inp.reshape(B * M, D)
    mf = objmask.reshape(B * M, 1)
    vals, ne = _tc_call(x, mf, objmask, W_act, b_act.reshape(1, NUM_ACT),
                        W_ne, b_ne.reshape(1, NUM_CLASSES))
    idx = AAidxs_tgts.astype(jnp.int32)
    return _sc_scatter(vals, idx, ne)[:, :NUM_CLASSES]
